# baseline (speedup 1.0000x reference)
import jax
import jax.numpy as jnp
from jax import lax
from jax.experimental import pallas as pl
from jax.experimental.pallas import tpu as pltpu
from jax.experimental.pallas import tpu_sc as plsc

TOKENS = 32768
D = 2048
B = 16

T_TC = 17408
T_SC = TOKENS - T_TC
BLK = 1024

NC = 2
NS = 16
LANES = 16
NSTRIPE = 16
CPW = D // NSTRIPE
HREG = CPW // LANES
TPH = T_SC // 2
CHUNK = 256
NCHUNK = TPH // CHUNK


NBUF = 2


def _pool_sc_body(hid_hbm, lens_hbm, out_hbm, buf0_v, buf1_v, acc_v,
                  lens_v, sem0, sem1):
    c = lax.axis_index("c")
    s = lax.axis_index("s")
    base_t = T_TC + c * TPH
    cbase = s * CPW

    def gather(ci, buf, sem):
        c0 = base_t + ci * CHUNK
        pltpu.async_copy(
            hid_hbm.at[pl.ds(c0, CHUNK), pl.ds(cbase, CPW)], buf, sem)

    def wait(buf, sem):
        pltpu.make_async_copy(
            hid_hbm.at[pl.ds(base_t, CHUNK), pl.ds(cbase, CPW)], buf,
            sem).wait()

    bufs = [buf0_v, buf1_v]
    sems = [sem0, sem1]
    for k in range(NBUF - 1):
        gather(k, bufs[k], sems[k])

    def zero_body(k, carry):
        acc_v[k // HREG, pl.ds((k % HREG) * LANES, LANES)] = (
            jnp.zeros((LANES,), jnp.float32))
        return carry

    lax.fori_loop(0, B * HREG, zero_body, 0)

    pltpu.sync_copy(lens_hbm, lens_v)
    ends = plsc.cumsum(lens_v[...])
    lane = lax.iota(jnp.int32, LANES)

    def process(ci, buf):
        c0 = base_t + ci * CHUNK

        def seg_body(b, prev_end):
            e_b = jnp.max(jnp.where(lane == b, ends, 0))
            lo = jnp.maximum(prev_end, c0) - c0
            hi = jnp.minimum(e_b, c0 + CHUNK) - c0

            @pl.when(hi > lo)
            def _():
                def tok_body(jt, regs):
                    return tuple(
                        regs[h] + buf[jt, pl.ds(h * LANES, LANES)]
                        for h in range(HREG))

                init = tuple(
                    acc_v[b, pl.ds(h * LANES, LANES)] for h in range(HREG))
                regs = lax.fori_loop(lo, hi, tok_body, init)
                for h in range(HREG):
                    acc_v[b, pl.ds(h * LANES, LANES)] = regs[h]

            return e_b

        lax.fori_loop(0, B, seg_body, jnp.int32(0))

    def group_body(jp, carry):
        for u in range(NBUF):
            ci = jp * NBUF + u
            pre = ci + NBUF - 1

            @pl.when(pre < NCHUNK)
            def _():
                gather(pre, bufs[(NBUF - 1 + u) % NBUF],
                       sems[(NBUF - 1 + u) % NBUF])

            wait(bufs[u], sems[u])
            process(ci, bufs[u])
        return carry

    ngroup = NCHUNK // NBUF
    lax.fori_loop(0, ngroup, group_body, 0)
    for ci in range(ngroup * NBUF, NCHUNK):
        u = ci % NBUF
        wait(bufs[u], sems[u])
        process(ci, bufs[u])
    pltpu.sync_copy(acc_v, out_hbm.at[c, :, pl.ds(cbase, CPW)])


def _pool_sc(hidden_states, prompt_lens):
    mesh = plsc.VectorSubcoreMesh(core_axis_name="c", subcore_axis_name="s")
    return pl.kernel(
        _pool_sc_body,
        out_type=jax.ShapeDtypeStruct((NC, B, D), jnp.float32),
        mesh=mesh,
        scratch_types=[
            pltpu.VMEM((CHUNK, CPW), jnp.float32),
            pltpu.VMEM((CHUNK, CPW), jnp.float32),
            pltpu.VMEM((B, CPW), jnp.float32),
            pltpu.VMEM((B,), jnp.int32),
            pltpu.SemaphoreType.DMA,
            pltpu.SemaphoreType.DMA,
        ],
        compiler_params=pltpu.CompilerParams(needs_layout_passes=False),
    )(hidden_states, prompt_lens)


def _pool_tc_body(lens_row_ref, x_ref, out_ref):
    i = pl.program_id(0)
    lens = lens_row_ref[...]
    tri = (jax.lax.broadcasted_iota(jnp.int32, (B, B), 0)
           <= jax.lax.broadcasted_iota(jnp.int32, (B, B), 1)).astype(jnp.float32)
    ends = jax.lax.dot_general(lens, tri, (((1,), (0,)), ((), ())),
                               precision=jax.lax.Precision.HIGHEST,
                               preferred_element_type=jnp.float32)
    starts = ends - lens
    rows = (i * BLK
            + jax.lax.broadcasted_iota(jnp.int32, (BLK, 1), 0)).astype(jnp.float32)
    oh = ((rows >= starts) & (rows < ends)).astype(jnp.float32)
    part = jax.lax.dot_general(
        oh, x_ref[...], (((0,), (0,)), ((), ())),
        precision=jax.lax.Precision.HIGHEST,
        preferred_element_type=jnp.float32)

    @pl.when(i == 0)
    def _():
        out_ref[...] = part

    @pl.when(i > 0)
    def _():
        out_ref[...] += part


def _pool_tc(hidden_states, lens_row):
    return pl.pallas_call(
        _pool_tc_body,
        grid=(T_TC // BLK,),
        in_specs=[
            pl.BlockSpec((1, B), lambda i: (0, 0)),
            pl.BlockSpec((BLK, D), lambda i: (i, 0)),
        ],
        out_specs=pl.BlockSpec((B, D), lambda i: (0, 0)),
        out_shape=jax.ShapeDtypeStruct((B, D), jnp.float32),
    )(lens_row, hidden_states)


def _finish_body(lens_col_ref, tc_ref, sc_ref, out_ref):
    total = tc_ref[...] + sc_ref[0] + sc_ref[1]
    pooled = total / lens_col_ref[...]
    ss = jnp.sum(pooled * pooled, axis=1, keepdims=True)
    out_ref[...] = pooled / jnp.maximum(jnp.sqrt(ss), 1e-12)


def kernel(hidden_states, prompt_lens):
    lens_row = prompt_lens.reshape(1, B).astype(jnp.float32)
    lens_col = prompt_lens.reshape(B, 1).astype(jnp.float32)
    sc_parts = _pool_sc(hidden_states, prompt_lens)
    tc_part = _pool_tc(hidden_states, lens_row)
    return pl.pallas_call(
        _finish_body,
        out_shape=jax.ShapeDtypeStruct((B, D), jnp.float32),
    )(lens_col, tc_part, sc_parts)


# speedup vs baseline: 1.0214x; 1.0214x over previous
import jax
import jax.numpy as jnp
from jax import lax
from jax.experimental import pallas as pl
from jax.experimental.pallas import tpu as pltpu
from jax.experimental.pallas import tpu_sc as plsc

TOKENS = 32768
D = 2048
B = 16

T_TC = 16384
T_SC = TOKENS - T_TC
BLK = 1024

NC = 2
NS = 16
LANES = 16
NSTRIPE = 16
CPW = D // NSTRIPE
HREG = CPW // LANES
TPH = T_SC // 2
CHUNK = 256
NCHUNK = TPH // CHUNK


NBUF = 2


def _pool_sc_body(hid_hbm, lens_hbm, out_hbm, buf0_v, buf1_v, acc_v,
                  lens_v, sem0, sem1):
    c = lax.axis_index("c")
    s = lax.axis_index("s")
    base_t = T_TC + c * TPH
    cbase = s * CPW

    def gather(ci, buf, sem):
        c0 = base_t + ci * CHUNK
        pltpu.async_copy(
            hid_hbm.at[pl.ds(c0, CHUNK), pl.ds(cbase, CPW)], buf, sem)

    def wait(buf, sem):
        pltpu.make_async_copy(
            hid_hbm.at[pl.ds(base_t, CHUNK), pl.ds(cbase, CPW)], buf,
            sem).wait()

    bufs = [buf0_v, buf1_v]
    sems = [sem0, sem1]
    for k in range(NBUF - 1):
        gather(k, bufs[k], sems[k])

    def zero_body(k, carry):
        acc_v[k // HREG, pl.ds((k % HREG) * LANES, LANES)] = (
            jnp.zeros((LANES,), jnp.float32))
        return carry

    lax.fori_loop(0, B * HREG, zero_body, 0)

    pltpu.sync_copy(lens_hbm, lens_v)
    ends = plsc.cumsum(lens_v[...])
    lane = lax.iota(jnp.int32, LANES)

    def seg_end(b):
        return jnp.max(jnp.where(lane == b, ends, 0))

    def process(ci, buf, st):
        c0 = base_t + ci * CHUNK
        c1 = c0 + CHUNK

        def cond(s):
            return s[0] < c1

        def span_body(s):
            pos, seg, end_b = s
            hi = jnp.minimum(end_b, c1)

            def tok_body(jt, regs):
                return tuple(
                    regs[h] + buf[jt, pl.ds(h * LANES, LANES)]
                    for h in range(HREG))

            init = tuple(
                acc_v[seg, pl.ds(h * LANES, LANES)] for h in range(HREG))
            regs = lax.fori_loop(pos - c0, hi - c0, tok_body, init)
            for h in range(HREG):
                acc_v[seg, pl.ds(h * LANES, LANES)] = regs[h]

            adv = end_b <= c1
            seg2 = jnp.where(adv, seg + 1, seg)
            end2 = jnp.where(adv, seg_end(seg2), end_b)
            return (hi, seg2, end2)

        out = lax.while_loop(cond, span_body, (c0, st[0], st[1]))
        return (out[1], out[2])

    def group_body(jp, st):
        for u in range(NBUF):
            ci = jp * NBUF + u
            pre = ci + NBUF - 1

            @pl.when(pre < NCHUNK)
            def _():
                gather(pre, bufs[(NBUF - 1 + u) % NBUF],
                       sems[(NBUF - 1 + u) % NBUF])

            wait(bufs[u], sems[u])
            st = process(ci, bufs[u], st)
        return st

    seg0 = jnp.sum((ends <= base_t).astype(jnp.int32))
    st = (seg0, seg_end(seg0))
    ngroup = NCHUNK // NBUF
    st = lax.fori_loop(0, ngroup, group_body, st)
    for ci in range(ngroup * NBUF, NCHUNK):
        u = ci % NBUF
        wait(bufs[u], sems[u])
        st = process(ci, bufs[u], st)
    pltpu.sync_copy(acc_v, out_hbm.at[c, :, pl.ds(cbase, CPW)])


def _pool_sc(hidden_states, prompt_lens):
    mesh = plsc.VectorSubcoreMesh(core_axis_name="c", subcore_axis_name="s")
    return pl.kernel(
        _pool_sc_body,
        out_type=jax.ShapeDtypeStruct((NC, B, D), jnp.float32),
        mesh=mesh,
        scratch_types=[
            pltpu.VMEM((CHUNK, CPW), jnp.float32),
            pltpu.VMEM((CHUNK, CPW), jnp.float32),
            pltpu.VMEM((B, CPW), jnp.float32),
            pltpu.VMEM((B,), jnp.int32),
            pltpu.SemaphoreType.DMA,
            pltpu.SemaphoreType.DMA,
        ],
        compiler_params=pltpu.CompilerParams(needs_layout_passes=False),
    )(hidden_states, prompt_lens)


def _pool_tc_body(lens_row_ref, x_ref, out_ref):
    i = pl.program_id(0)
    lens = lens_row_ref[...]
    tri = (jax.lax.broadcasted_iota(jnp.int32, (B, B), 0)
           <= jax.lax.broadcasted_iota(jnp.int32, (B, B), 1)).astype(jnp.float32)
    ends = jax.lax.dot_general(lens, tri, (((1,), (0,)), ((), ())),
                               precision=jax.lax.Precision.HIGHEST,
                               preferred_element_type=jnp.float32)
    starts = ends - lens
    rows = (i * BLK
            + jax.lax.broadcasted_iota(jnp.int32, (BLK, 1), 0)).astype(jnp.float32)
    oh = ((rows >= starts) & (rows < ends)).astype(jnp.float32)
    part = jax.lax.dot_general(
        oh, x_ref[...], (((0,), (0,)), ((), ())),
        precision=jax.lax.Precision.HIGHEST,
        preferred_element_type=jnp.float32)

    @pl.when(i == 0)
    def _():
        out_ref[...] = part

    @pl.when(i > 0)
    def _():
        out_ref[...] += part


def _pool_tc(hidden_states, lens_row):
    return pl.pallas_call(
        _pool_tc_body,
        grid=(T_TC // BLK,),
        in_specs=[
            pl.BlockSpec((1, B), lambda i: (0, 0)),
            pl.BlockSpec((BLK, D), lambda i: (i, 0)),
        ],
        out_specs=pl.BlockSpec((B, D), lambda i: (0, 0)),
        out_shape=jax.ShapeDtypeStruct((B, D), jnp.float32),
    )(lens_row, hidden_states)


def _finish_body(lens_col_ref, tc_ref, sc_ref, out_ref):
    total = tc_ref[...] + sc_ref[0] + sc_ref[1]
    pooled = total / lens_col_ref[...]
    ss = jnp.sum(pooled * pooled, axis=1, keepdims=True)
    out_ref[...] = pooled / jnp.maximum(jnp.sqrt(ss), 1e-12)


def kernel(hidden_states, prompt_lens):
    lens_row = prompt_lens.reshape(1, B).astype(jnp.float32)
    lens_col = prompt_lens.reshape(B, 1).astype(jnp.float32)
    sc_parts = _pool_sc(hidden_states, prompt_lens)
    tc_part = _pool_tc(hidden_states, lens_row)
    return pl.pallas_call(
        _finish_body,
        out_shape=jax.ShapeDtypeStruct((B, D), jnp.float32),
    )(lens_col, tc_part, sc_parts)
